# baseline (speedup 1.0000x reference)
import jax
import jax.numpy as jnp
from jax.experimental import pallas as pl
from jax.experimental.pallas import tpu as pltpu

_N_CHUNKS = 8


def _copy_body(src_hbm, out_hbm, sems):
    rows = src_hbm.shape[0]
    chunk = rows // _N_CHUNKS
    for i in range(_N_CHUNKS):
        pltpu.make_async_copy(
            src_hbm.at[pl.ds(i * chunk, chunk), :],
            out_hbm.at[pl.ds(i * chunk, chunk), :],
            sems.at[i],
        ).start()
    for i in range(_N_CHUNKS):
        pltpu.make_async_copy(
            src_hbm.at[pl.ds(i * chunk, chunk), :],
            out_hbm.at[pl.ds(i * chunk, chunk), :],
            sems.at[i],
        ).wait()


def kernel(input_ids, pos_table):
    seq_len = input_ids.shape[1]
    rows, dim = pos_table.shape
    assert seq_len == rows
    return pl.pallas_call(
        _copy_body,
        in_specs=[pl.BlockSpec(memory_space=pltpu.MemorySpace.HBM)],
        out_specs=pl.BlockSpec(memory_space=pltpu.MemorySpace.HBM),
        out_shape=jax.ShapeDtypeStruct((seq_len, dim), pos_table.dtype),
        scratch_shapes=[pltpu.SemaphoreType.DMA((_N_CHUNKS,))],
    )(pos_table)


# speedup vs baseline: 48.3407x; 48.3407x over previous
import jax
import jax.numpy as jnp
from jax.experimental import pallas as pl
from jax.experimental.pallas import tpu as pltpu

_BLOCK_ROWS = 2048


def _copy_body(src_ref, out_ref):
    out_ref[...] = src_ref[...]


def kernel(input_ids, pos_table):
    seq_len = input_ids.shape[1]
    rows, dim = pos_table.shape
    assert seq_len == rows
    grid = (rows // _BLOCK_ROWS,)
    return pl.pallas_call(
        _copy_body,
        grid=grid,
        in_specs=[pl.BlockSpec((_BLOCK_ROWS, dim), lambda i: (i, 0))],
        out_specs=pl.BlockSpec((_BLOCK_ROWS, dim), lambda i: (i, 0)),
        out_shape=jax.ShapeDtypeStruct((seq_len, dim), pos_table.dtype),
        compiler_params=pltpu.CompilerParams(
            dimension_semantics=("parallel",),
        ),
    )(pos_table)
